# baseline (speedup 1.0000x reference)
import functools

import jax
import jax.numpy as jnp
from jax import lax
from jax.experimental import pallas as pl
from jax.experimental.pallas import tpu as pltpu
from jax.experimental.pallas import tpu_sc as plsc

N_NODES = 10000
N_EDGES = 160000
D = 256
EPS = 1e-6

NC = 2
NS = 16
NW = NC * NS


def _sc_gather_rows(table, idx, *, chunk=200):
    n_per_w = N_EDGES // NW
    n_chunks = n_per_w // chunk
    mesh = plsc.VectorSubcoreMesh(
        core_axis_name="c", subcore_axis_name="s", num_cores=NC, num_subcores=NS)

    @functools.partial(
        pl.kernel,
        out_type=jax.ShapeDtypeStruct((N_EDGES, D), jnp.float32),
        mesh=mesh,
        scratch_types=[
            pltpu.VMEM((chunk,), jnp.int32),
            pltpu.VMEM((chunk, D), jnp.float32),
            pltpu.SemaphoreType.DMA,
        ],
    )
    def k(table_hbm, idx_hbm, out_hbm, idx_v, rows_v, sem):
        wid = lax.axis_index("s") * NC + lax.axis_index("c")
        base = wid * n_per_w
        for c in range(n_chunks):
            off = pl.multiple_of(base + c * chunk, 8)
            pltpu.sync_copy(idx_hbm.at[pl.ds(off, chunk)], idx_v)
            pltpu.async_copy(table_hbm.at[idx_v], rows_v, sem).wait()
            pltpu.sync_copy(rows_v, out_hbm.at[pl.ds(off, chunk)])

    return k(table, idx)


def _mlp1_body(hvi, he, hvj, w1a, w1b, w1c, w2t, w3t, b1, b2, b3, out):
    dh = jnp.dot(hvi[...], w1a[...], preferred_element_type=jnp.float32)
    dh += jnp.dot(he[...], w1b[...], preferred_element_type=jnp.float32)
    dh += jnp.dot(hvj[...], w1c[...], preferred_element_type=jnp.float32)
    dh += b1[...]
    dh = jnp.dot(dh, w2t[...], preferred_element_type=jnp.float32) + b2[...]
    dh = jnp.maximum(dh, 0.0)
    out[...] = jnp.dot(dh, w3t[...], preferred_element_type=jnp.float32) + b3[...]


def _run_mlp1(hvi, he, hvj, W1, b1, W2, b2, W3, b3, *, block=2000):
    W1t = W1.T
    w1a, w1b, w1c = W1t[0:D], W1t[D:2 * D], W1t[2 * D:3 * D]
    grid = (N_EDGES // block,)
    row_spec = pl.BlockSpec((block, D), lambda i: (i, 0))
    full = lambda s: pl.BlockSpec(s, lambda i: (0,) * len(s))
    return pl.pallas_call(
        _mlp1_body,
        grid=grid,
        in_specs=[row_spec, row_spec, row_spec,
                  full((D, D)), full((D, D)), full((D, D)),
                  full((D, D)), full((D, D)),
                  full((1, D)), full((1, D)), full((1, D))],
        out_specs=row_spec,
        out_shape=jax.ShapeDtypeStruct((N_EDGES, D), jnp.float32),
    )(hvi, he, hvj, w1a, w1b, w1c, W2.T, W3.T,
      b1.reshape(1, D), b2.reshape(1, D), b3.reshape(1, D))


def _normalize(x, gain, bias):
    mu = jnp.mean(x, axis=-1, keepdims=True)
    xc = x - mu
    var = jnp.sum(xc * xc, axis=-1, keepdims=True) * (1.0 / (D - 1))
    sigma = jnp.sqrt(var + EPS)
    return gain * xc / (sigma + EPS) + bias


def _tail_body(dh, dhrev, mask, he, wf1t, wf2t, bf1, bf2, g0, bn0, g1, bn1, out):
    m = mask[...]
    d = dh[...]
    merged = d + m * (0.5 * (dhrev[...] + d) - d)
    x = _normalize(he[...] + merged, g0[...], bn0[...])
    y = jnp.dot(x, wf1t[...], preferred_element_type=jnp.float32) + bf1[...]
    z = jnp.dot(y, wf2t[...], preferred_element_type=jnp.float32) + bf2[...]
    out[...] = _normalize(x + z, g1[...], bn1[...])


def _run_tail(dh, dhrev, mask, he, Wf1, bf1, Wf2, bf2, g0, bn0, g1, bn1, *, block=2000):
    grid = (N_EDGES // block,)
    row_spec = pl.BlockSpec((block, D), lambda i: (i, 0))
    mask_spec = pl.BlockSpec((block, 1), lambda i: (i, 0))
    full = lambda s: pl.BlockSpec(s, lambda i: (0,) * len(s))
    return pl.pallas_call(
        _tail_body,
        grid=grid,
        in_specs=[row_spec, row_spec, mask_spec, row_spec,
                  full((D, 2 * D)), full((2 * D, D)),
                  full((1, 2 * D)), full((1, D)),
                  full((1, D)), full((1, D)), full((1, D)), full((1, D))],
        out_specs=row_spec,
        out_shape=jax.ShapeDtypeStruct((N_EDGES, D), jnp.float32),
    )(dh, dhrev, mask, he, Wf1.T, Wf2.T,
      bf1.reshape(1, 2 * D), bf2.reshape(1, D),
      g0.reshape(1, D), bn0.reshape(1, D), g1.reshape(1, D), bn1.reshape(1, D))


HALF_KEYS = (N_NODES * N_NODES) // 2
TABLE_N = N_NODES * N_NODES
TABLE_PAD = TABLE_N + 8 * NW
E_BITS = 18
_ESH = N_EDGES // NS


def _mesh():
    return plsc.VectorSubcoreMesh(
        core_axis_name="c", subcore_axis_name="s", num_cores=NC, num_subcores=NS)


def _sc_partner_phase1(src, dst):
    n_vr = _ESH // 16

    @functools.partial(
        pl.kernel,
        out_type=(jax.ShapeDtypeStruct((TABLE_PAD,), jnp.int32),
                  jax.ShapeDtypeStruct((N_EDGES + 16,), jnp.int32),
                  jax.ShapeDtypeStruct((N_EDGES + 16,), jnp.int32)),
        mesh=_mesh(),
        scratch_types=[
            pltpu.VMEM((_ESH,), jnp.int32),
            pltpu.VMEM((_ESH,), jnp.int32),
            pltpu.VMEM((_ESH,), jnp.int32),
            pltpu.VMEM((_ESH,), jnp.int32),
            pltpu.VMEM((_ESH,), jnp.int32),
            pltpu.VMEM((_ESH,), jnp.int32),
            pltpu.SemaphoreType.DMA,
        ],
    )
    def k1(src_hbm, dst_hbm, t_hbm, kh_hbm, rh_hbm,
           kb, rb, alive, widx, wval, tb, sem):
        core = lax.axis_index("c")
        sid = lax.axis_index("s")
        base = sid * _ESH
        dummy = TABLE_N + (sid * NC + core) * 8
        kmin = core * HALF_KEYS

        pltpu.sync_copy(src_hbm.at[pl.ds(base, _ESH)], kb)
        pltpu.sync_copy(dst_hbm.at[pl.ds(base, _ESH)], rb)

        def prep(i, _):
            sl = pl.ds(i * 16, 16)
            s = kb[sl]
            d = rb[sl]
            kv = s * N_NODES + d
            kb[sl] = kv
            rb[sl] = d * N_NODES + s
            inhalf = (kv >= kmin) & (kv < kmin + HALF_KEYS)
            alive[sl] = jnp.where(inhalf, 1, 0)
            widx[sl] = jnp.where(inhalf, kv, dummy)
            wval[sl] = jnp.full((16,), -1, jnp.int32)
            return 0

        lax.fori_loop(0, n_vr, prep, 0)

        @pl.when(core == 0)
        def _():
            pltpu.sync_copy(kb, kh_hbm.at[pl.ds(base, _ESH)])
            pltpu.sync_copy(rb, rh_hbm.at[pl.ds(base, _ESH)])

        pltpu.async_copy(wval, t_hbm.at[widx], sem).wait()
        plsc.subcore_barrier()

        def rnd(r, _):
            b = (E_BITS - 1) - r

            def mk(i, _):
                sl = pl.ds(i * 16, 16)
                e = jnp.full((16,), base + i * 16, jnp.int32) + lax.iota(jnp.int32, 16)
                bit = lax.shift_right_logical(e, b) & 1
                w = (alive[sl] == 1) & (bit == 0)
                widx[sl] = jnp.where(w, kb[sl], dummy)
                wval[sl] = jnp.full((16,), 0, jnp.int32) + r
                return 0

            lax.fori_loop(0, n_vr, mk, 0)
            pltpu.async_copy(wval, t_hbm.at[widx], sem).wait()
            plsc.subcore_barrier()
            pltpu.async_copy(t_hbm.at[kb], tb, sem).wait()

            def upd(i, _):
                sl = pl.ds(i * 16, 16)
                e = jnp.full((16,), base + i * 16, jnp.int32) + lax.iota(jnp.int32, 16)
                bit = lax.shift_right_logical(e, b) & 1
                die = (alive[sl] == 1) & (tb[sl] == r) & (bit == 1)
                alive[sl] = jnp.where(die, 0, alive[sl])
                return 0

            lax.fori_loop(0, n_vr, upd, 0)
            plsc.subcore_barrier()
            return 0

        lax.fori_loop(0, E_BITS, rnd, 0)

        def fin(i, _):
            sl = pl.ds(i * 16, 16)
            e = jnp.full((16,), base + i * 16, jnp.int32) + lax.iota(jnp.int32, 16)
            widx[sl] = jnp.where(alive[sl] == 1, kb[sl], dummy)
            wval[sl] = e
            return 0

        lax.fori_loop(0, n_vr, fin, 0)
        pltpu.async_copy(wval, t_hbm.at[widx], sem).wait()

    return k1(src, dst)


def _sc_partner_phase2(tbl, khash, rhash):
    n_per_w = N_EDGES // NW
    n_vr = (n_per_w + 15) // 16
    npad = n_vr * 16

    @functools.partial(
        pl.kernel,
        out_type=(jax.ShapeDtypeStruct((N_EDGES,), jnp.int32),
                  jax.ShapeDtypeStruct((N_EDGES,), jnp.float32)),
        mesh=_mesh(),
        scratch_types=[
            pltpu.VMEM((npad,), jnp.int32),
            pltpu.VMEM((npad,), jnp.int32),
            pltpu.VMEM((npad,), jnp.int32),
            pltpu.VMEM((npad,), jnp.int32),
            pltpu.VMEM((npad,), jnp.float32),
            pltpu.SemaphoreType.DMA,
        ],
    )
    def k2(t_hbm, kh_hbm, rh_hbm, p_hbm, m_hbm, rv, pv, pc, kv2, mf, sem):
        wid = lax.axis_index("s") * NC + lax.axis_index("c")
        base = wid * n_per_w
        pltpu.sync_copy(rh_hbm.at[pl.ds(base, npad)], rv)

        def clampi(i, _):
            sl = pl.ds(i * 16, 16)
            rv[sl] = jnp.clip(rv[sl], 0, TABLE_N - 1)
            return 0

        lax.fori_loop(0, n_vr, clampi, 0)
        pltpu.async_copy(t_hbm.at[rv], pv, sem).wait()

        def clampp(i, _):
            sl = pl.ds(i * 16, 16)
            pc[sl] = jnp.clip(pv[sl], 0, N_EDGES - 1)
            return 0

        lax.fori_loop(0, n_vr, clampp, 0)
        pltpu.async_copy(kh_hbm.at[pc], kv2, sem).wait()

        def verify(i, _):
            sl = pl.ds(i * 16, 16)
            hit = kv2[sl] == rv[sl]
            mf[sl] = jnp.where(hit, 1.0, 0.0).astype(jnp.float32)
            return 0

        lax.fori_loop(0, n_vr, verify, 0)
        pltpu.sync_copy(pc.at[pl.ds(0, n_per_w)], p_hbm.at[pl.ds(base, n_per_w)])
        pltpu.sync_copy(mf.at[pl.ds(0, n_per_w)], m_hbm.at[pl.ds(base, n_per_w)])

    return k2(tbl, khash, rhash)


def kernel(h_V, edge_index, h_E, W1, b1, W2, b2, W3, b3,
           Wf1, bf1, Wf2, bf2, g0, bn0, g1, bn1):
    src = edge_index[0]
    dst = edge_index[1]
    tbl, khash, rhash = _sc_partner_phase1(src, dst)
    partner, maskf = _sc_partner_phase2(tbl, khash, rhash)
    hvi = _sc_gather_rows(h_V, src)
    hvj = _sc_gather_rows(h_V, dst)
    dh = _run_mlp1(hvi, h_E, hvj, W1, b1, W2, b2, W3, b3)
    dhrev = _sc_gather_rows(dh, partner)
    mask = maskf.reshape(N_EDGES, 1)
    return _run_tail(dh, dhrev, mask, h_E, Wf1, bf1, Wf2, bf2, g0, bn0, g1, bn1)


# speedup vs baseline: 29.9354x; 29.9354x over previous
import functools

import jax
import jax.numpy as jnp
from jax import lax
from jax.experimental import pallas as pl
from jax.experimental.pallas import tpu as pltpu
from jax.experimental.pallas import tpu_sc as plsc

N_NODES = 10000
N_EDGES = 160000
D = 256
EPS = 1e-6

NC = 2
NS = 16
NW = NC * NS


def _sc_gather_rows(table, idx, *, chunk=200):
    n_per_w = N_EDGES // NW
    n_chunks = n_per_w // chunk
    mesh = plsc.VectorSubcoreMesh(
        core_axis_name="c", subcore_axis_name="s", num_cores=NC, num_subcores=NS)

    @functools.partial(
        pl.kernel,
        out_type=jax.ShapeDtypeStruct((N_EDGES, D), jnp.float32),
        mesh=mesh,
        scratch_types=[
            pltpu.VMEM((chunk,), jnp.int32),
            pltpu.VMEM((chunk, D), jnp.float32),
            pltpu.SemaphoreType.DMA,
        ],
    )
    def k(table_hbm, idx_hbm, out_hbm, idx_v, rows_v, sem):
        wid = lax.axis_index("s") * NC + lax.axis_index("c")
        base = wid * n_per_w
        for c in range(n_chunks):
            off = pl.multiple_of(base + c * chunk, 8)
            pltpu.sync_copy(idx_hbm.at[pl.ds(off, chunk)], idx_v)
            pltpu.async_copy(table_hbm.at[idx_v], rows_v, sem).wait()
            pltpu.sync_copy(rows_v, out_hbm.at[pl.ds(off, chunk)])

    return k(table, idx)


def _mlp1_body(hvi, he, hvj, w1a, w1b, w1c, w2t, w3t, b1, b2, b3, out):
    dh = jnp.dot(hvi[...], w1a[...], preferred_element_type=jnp.float32)
    dh += jnp.dot(he[...], w1b[...], preferred_element_type=jnp.float32)
    dh += jnp.dot(hvj[...], w1c[...], preferred_element_type=jnp.float32)
    dh += b1[...]
    dh = jnp.dot(dh, w2t[...], preferred_element_type=jnp.float32) + b2[...]
    dh = jnp.maximum(dh, 0.0)
    out[...] = jnp.dot(dh, w3t[...], preferred_element_type=jnp.float32) + b3[...]


def _run_mlp1(hvi, he, hvj, W1, b1, W2, b2, W3, b3, *, block=2000):
    W1t = W1.T
    w1a, w1b, w1c = W1t[0:D], W1t[D:2 * D], W1t[2 * D:3 * D]
    grid = (N_EDGES // block,)
    row_spec = pl.BlockSpec((block, D), lambda i: (i, 0))
    full = lambda s: pl.BlockSpec(s, lambda i: (0,) * len(s))
    return pl.pallas_call(
        _mlp1_body,
        grid=grid,
        in_specs=[row_spec, row_spec, row_spec,
                  full((D, D)), full((D, D)), full((D, D)),
                  full((D, D)), full((D, D)),
                  full((1, D)), full((1, D)), full((1, D))],
        out_specs=row_spec,
        out_shape=jax.ShapeDtypeStruct((N_EDGES, D), jnp.float32),
    )(hvi, he, hvj, w1a, w1b, w1c, W2.T, W3.T,
      b1.reshape(1, D), b2.reshape(1, D), b3.reshape(1, D))


def _normalize(x, gain, bias):
    mu = jnp.mean(x, axis=-1, keepdims=True)
    xc = x - mu
    var = jnp.sum(xc * xc, axis=-1, keepdims=True) * (1.0 / (D - 1))
    sigma = jnp.sqrt(var + EPS)
    return gain * xc / (sigma + EPS) + bias


def _tail_body(dh, dhrev, mask, he, wf1t, wf2t, bf1, bf2, g0, bn0, g1, bn1, out):
    m = mask[...]
    d = dh[...]
    merged = d + m * (0.5 * (dhrev[...] + d) - d)
    x = _normalize(he[...] + merged, g0[...], bn0[...])
    y = jnp.dot(x, wf1t[...], preferred_element_type=jnp.float32) + bf1[...]
    z = jnp.dot(y, wf2t[...], preferred_element_type=jnp.float32) + bf2[...]
    out[...] = _normalize(x + z, g1[...], bn1[...])


def _run_tail(dh, dhrev, mask, he, Wf1, bf1, Wf2, bf2, g0, bn0, g1, bn1, *, block=2000):
    grid = (N_EDGES // block,)
    row_spec = pl.BlockSpec((block, D), lambda i: (i, 0))
    mask_spec = pl.BlockSpec((block, 1), lambda i: (i, 0))
    full = lambda s: pl.BlockSpec(s, lambda i: (0,) * len(s))
    return pl.pallas_call(
        _tail_body,
        grid=grid,
        in_specs=[row_spec, row_spec, mask_spec, row_spec,
                  full((D, 2 * D)), full((2 * D, D)),
                  full((1, 2 * D)), full((1, D)),
                  full((1, D)), full((1, D)), full((1, D)), full((1, D))],
        out_specs=row_spec,
        out_shape=jax.ShapeDtypeStruct((N_EDGES, D), jnp.float32),
    )(dh, dhrev, mask, he, Wf1.T, Wf2.T,
      bf1.reshape(1, 2 * D), bf2.reshape(1, D),
      g0.reshape(1, D), bn0.reshape(1, D), g1.reshape(1, D), bn1.reshape(1, D))


HALF_KEYS = (N_NODES * N_NODES) // 2
TABLE_N = N_NODES * N_NODES
PAD_N = 1 << 20
TABLE_PAD = TABLE_N + PAD_N
E_BITS = 18
_ESH = N_EDGES // NS


def _mesh():
    return plsc.VectorSubcoreMesh(
        core_axis_name="c", subcore_axis_name="s", num_cores=NC, num_subcores=NS)


def _sc_partner_phase1(src, dst):
    n_vr = _ESH // 16

    @functools.partial(
        pl.kernel,
        out_type=(jax.ShapeDtypeStruct((TABLE_PAD,), jnp.int32),
                  jax.ShapeDtypeStruct((N_EDGES + 16,), jnp.int32),
                  jax.ShapeDtypeStruct((N_EDGES + 16,), jnp.int32)),
        mesh=_mesh(),
        scratch_types=[
            pltpu.VMEM((_ESH,), jnp.int32),
            pltpu.VMEM((_ESH,), jnp.int32),
            pltpu.VMEM((_ESH,), jnp.int32),
            pltpu.VMEM((_ESH,), jnp.int32),
            pltpu.VMEM((_ESH,), jnp.int32),
            pltpu.VMEM((_ESH,), jnp.int32),
            pltpu.SemaphoreType.DMA,
        ],
    )
    def k1(src_hbm, dst_hbm, t_hbm, kh_hbm, rh_hbm,
           kb, rb, alive, widx, wval, tb, sem):
        core = lax.axis_index("c")
        sid = lax.axis_index("s")
        base = sid * _ESH
        kmin = core * HALF_KEYS

        pltpu.sync_copy(src_hbm.at[pl.ds(base, _ESH)], kb)
        pltpu.sync_copy(dst_hbm.at[pl.ds(base, _ESH)], rb)

        def prep(i, _):
            sl = pl.ds(i * 16, 16)
            s = kb[sl]
            d = rb[sl]
            kv = s * N_NODES + d
            e = jnp.full((16,), base + i * 16, jnp.int32) + lax.iota(jnp.int32, 16)
            dummy = TABLE_N + ((e * 8 + core) & (PAD_N - 1))
            kb[sl] = kv
            rb[sl] = d * N_NODES + s
            inhalf = (kv >= kmin) & (kv < kmin + HALF_KEYS)
            alive[sl] = jnp.where(inhalf, 1, 0)
            widx[sl] = jnp.where(inhalf, kv, dummy)
            wval[sl] = jnp.full((16,), -1, jnp.int32)
            return 0

        lax.fori_loop(0, n_vr, prep, 0)

        @pl.when(core == 0)
        def _():
            pltpu.sync_copy(kb, kh_hbm.at[pl.ds(base, _ESH)])
            pltpu.sync_copy(rb, rh_hbm.at[pl.ds(base, _ESH)])

        pltpu.async_copy(wval, t_hbm.at[widx], sem).wait()
        plsc.subcore_barrier()

        def rnd(r, _):
            b = (E_BITS - 1) - r

            def mk(i, _):
                sl = pl.ds(i * 16, 16)
                e = jnp.full((16,), base + i * 16, jnp.int32) + lax.iota(jnp.int32, 16)
                bit = lax.shift_right_logical(e, b) & 1
                w = (alive[sl] == 1) & (bit == 0)
                dummy = TABLE_N + ((e * 8 + core) & (PAD_N - 1))
                widx[sl] = jnp.where(w, kb[sl], dummy)
                wval[sl] = jnp.full((16,), 0, jnp.int32) + r
                return 0

            lax.fori_loop(0, n_vr, mk, 0)
            pltpu.async_copy(wval, t_hbm.at[widx], sem).wait()
            plsc.subcore_barrier()
            pltpu.async_copy(t_hbm.at[kb], tb, sem).wait()

            def upd(i, _):
                sl = pl.ds(i * 16, 16)
                e = jnp.full((16,), base + i * 16, jnp.int32) + lax.iota(jnp.int32, 16)
                bit = lax.shift_right_logical(e, b) & 1
                die = (alive[sl] == 1) & (tb[sl] == r) & (bit == 1)
                alive[sl] = jnp.where(die, 0, alive[sl])
                return 0

            lax.fori_loop(0, n_vr, upd, 0)
            plsc.subcore_barrier()
            return 0

        lax.fori_loop(0, E_BITS, rnd, 0)

        def fin(i, _):
            sl = pl.ds(i * 16, 16)
            e = jnp.full((16,), base + i * 16, jnp.int32) + lax.iota(jnp.int32, 16)
            dummy = TABLE_N + ((e * 8 + core) & (PAD_N - 1))
            widx[sl] = jnp.where(alive[sl] == 1, kb[sl], dummy)
            wval[sl] = e
            return 0

        lax.fori_loop(0, n_vr, fin, 0)
        pltpu.async_copy(wval, t_hbm.at[widx], sem).wait()

    return k1(src, dst)


def _sc_partner_phase2(tbl, khash, rhash):
    n_per_w = N_EDGES // NW
    n_vr = (n_per_w + 15) // 16
    npad = n_vr * 16

    @functools.partial(
        pl.kernel,
        out_type=(jax.ShapeDtypeStruct((N_EDGES,), jnp.int32),
                  jax.ShapeDtypeStruct((N_EDGES,), jnp.float32)),
        mesh=_mesh(),
        scratch_types=[
            pltpu.VMEM((npad,), jnp.int32),
            pltpu.VMEM((npad,), jnp.int32),
            pltpu.VMEM((npad,), jnp.int32),
            pltpu.VMEM((npad,), jnp.int32),
            pltpu.VMEM((npad,), jnp.float32),
            pltpu.SemaphoreType.DMA,
        ],
    )
    def k2(t_hbm, kh_hbm, rh_hbm, p_hbm, m_hbm, rv, pv, pc, kv2, mf, sem):
        wid = lax.axis_index("s") * NC + lax.axis_index("c")
        base = wid * n_per_w
        pltpu.sync_copy(rh_hbm.at[pl.ds(base, npad)], rv)

        def clampi(i, _):
            sl = pl.ds(i * 16, 16)
            rv[sl] = jnp.clip(rv[sl], 0, TABLE_N - 1)
            return 0

        lax.fori_loop(0, n_vr, clampi, 0)
        pltpu.async_copy(t_hbm.at[rv], pv, sem).wait()

        def clampp(i, _):
            sl = pl.ds(i * 16, 16)
            p = pv[sl]
            valid = (p >= 0) & (p < N_EDGES)
            e = jnp.full((16,), base + i * 16, jnp.int32) + lax.iota(jnp.int32, 16)
            pc[sl] = jnp.where(valid, p, e)
            return 0

        lax.fori_loop(0, n_vr, clampp, 0)
        pltpu.async_copy(kh_hbm.at[pc], kv2, sem).wait()

        def verify(i, _):
            sl = pl.ds(i * 16, 16)
            hit = kv2[sl] == rv[sl]
            mf[sl] = jnp.where(hit, 1.0, 0.0).astype(jnp.float32)
            return 0

        lax.fori_loop(0, n_vr, verify, 0)
        pltpu.sync_copy(pc.at[pl.ds(0, n_per_w)], p_hbm.at[pl.ds(base, n_per_w)])
        pltpu.sync_copy(mf.at[pl.ds(0, n_per_w)], m_hbm.at[pl.ds(base, n_per_w)])

    return k2(tbl, khash, rhash)


def kernel(h_V, edge_index, h_E, W1, b1, W2, b2, W3, b3,
           Wf1, bf1, Wf2, bf2, g0, bn0, g1, bn1):
    src = edge_index[0]
    dst = edge_index[1]
    tbl, khash, rhash = _sc_partner_phase1(src, dst)
    partner, maskf = _sc_partner_phase2(tbl, khash, rhash)
    hvi = _sc_gather_rows(h_V, src)
    hvj = _sc_gather_rows(h_V, dst)
    dh = _run_mlp1(hvi, h_E, hvj, W1, b1, W2, b2, W3, b3)
    dhrev = _sc_gather_rows(dh, partner)
    mask = maskf.reshape(N_EDGES, 1)
    return _run_tail(dh, dhrev, mask, h_E, Wf1, bf1, Wf2, bf2, g0, bn0, g1, bn1)
